# baseline (speedup 1.0000x reference)
import jax
import jax.numpy as jnp
from jax.experimental import pallas as pl
from jax.experimental.pallas import tpu as pltpu

B = 4096
D = 128
A = 1000
K = 16
C = B // K


def _body(sidx_ref, x_ref, b_ref, *rest):
    w_refs = rest[:K]
    o_ref = rest[K]
    i = pl.program_id(0)
    for k in range(K):
        bidx = sidx_ref[k * C + i]
        x_row = x_ref[0, pl.ds(k, 1), :]
        z = jax.lax.dot_general(x_row, w_refs[k][0], (((1,), (1,)), ((), ())),
                                preferred_element_type=jnp.float32)
        o_ref[0, pl.ds(k, 1), :] = z + b_ref[pl.ds(bidx, 1), :]


def kernel(x, a, weight, bias):
    idx = a[:, 0].astype(jnp.int32)
    sidx, perm = jax.lax.sort_key_val(idx, jnp.arange(B, dtype=jnp.int32))
    x_r = x[perm].reshape(K, C, D).transpose(1, 0, 2)

    def w_spec(k):
        return pl.BlockSpec(
            (1, D, D),
            lambda i, sidx_ref, _k=k: (sidx_ref[_k * C + i], 0, 0))

    z_r = pl.pallas_call(
        _body,
        grid_spec=pltpu.PrefetchScalarGridSpec(
            num_scalar_prefetch=1,
            grid=(C,),
            in_specs=[
                pl.BlockSpec((1, K, D), lambda i, s_: (i, 0, 0)),
                pl.BlockSpec((A, D), lambda i, s_: (0, 0)),
            ] + [w_spec(k) for k in range(K)],
            out_specs=pl.BlockSpec((1, K, D), lambda i, s_: (i, 0, 0)),
        ),
        out_shape=jax.ShapeDtypeStruct((C, K, D), jnp.float32),
    )(sidx, x_r, bias, *([weight] * K))

    z_s = z_r.transpose(1, 0, 2).reshape(B, D)
    return jnp.zeros((B, D), jnp.float32).at[perm].set(z_s)


# speedup vs baseline: 1.2804x; 1.2804x over previous
import jax
import jax.numpy as jnp
from jax.experimental import pallas as pl
from jax.experimental.pallas import tpu as pltpu

B = 4096
D = 128
A = 1000
G = 8
K = 16
NG = 1392
C = NG // K
P = NG * G


def _body(garr_ref, x_ref, b_ref, *rest):
    w_refs = rest[:K]
    o_ref = rest[K]
    i = pl.program_id(0)
    for k in range(K):
        bidx = garr_ref[k * C + i]
        xg = x_ref[k, 0]
        z = jax.lax.dot_general(xg, w_refs[k][0], (((1,), (1,)), ((), ())),
                                preferred_element_type=jnp.float32)
        o_ref[k, 0] = z + b_ref[pl.ds(bidx, 1), :]


def kernel(x, a, weight, bias):
    idx = a[:, 0].astype(jnp.int32)
    iota = jnp.arange(B, dtype=jnp.int32)
    sidx, perm = jax.lax.sort_key_val(idx, iota)

    starts = jnp.concatenate([jnp.ones((1,), jnp.bool_),
                              sidx[1:] != sidx[:-1]])
    seg_start = jax.lax.cummax(jnp.where(starts, iota, 0))
    pos_in_run = iota - seg_start
    new_group = starts | (pos_in_run % G == 0)
    g = jnp.cumsum(new_group.astype(jnp.int32)) - 1
    padded_pos = g * G + pos_in_run % G

    garr = jnp.zeros((NG,), jnp.int32).at[g].set(sidx)
    xsrc = jnp.zeros((P,), jnp.int32).at[padded_pos].set(perm)
    slot_of_row = jnp.zeros((B,), jnp.int32).at[perm].set(padded_pos)

    x_pad = jnp.take(x, xsrc, axis=0).reshape(K, C, G, D)

    def w_spec(k):
        return pl.BlockSpec(
            (1, D, D),
            lambda i, g_ref, _k=k: (g_ref[_k * C + i], 0, 0))

    z_pad = pl.pallas_call(
        _body,
        grid_spec=pltpu.PrefetchScalarGridSpec(
            num_scalar_prefetch=1,
            grid=(C,),
            in_specs=[
                pl.BlockSpec((K, 1, G, D), lambda i, g_: (0, i, 0, 0)),
                pl.BlockSpec((A, D), lambda i, g_: (0, 0)),
            ] + [w_spec(k) for k in range(K)],
            out_specs=pl.BlockSpec((K, 1, G, D), lambda i, g_: (0, i, 0, 0)),
        ),
        out_shape=jax.ShapeDtypeStruct((K, C, G, D), jnp.float32),
    )(garr, x_pad, bias, *([weight] * K))

    return jnp.take(z_pad.reshape(P, D), slot_of_row, axis=0)
